# Initial kernel scaffold; baseline (speedup 1.0000x reference)
import jax
import jax.numpy as jnp
from jax.experimental import pallas as pl


def kernel(img, features, segments, centroid_coords, W_pos, b_pos):
    raise NotImplementedError("write your pallas kernel here")


# speedup vs baseline: 1.4294x; 1.4294x over previous
import functools

import jax
import jax.numpy as jnp
from jax import lax
from jax.experimental import pallas as pl
from jax.experimental.pallas import tpu as pltpu
from jax.experimental.pallas import tpu_sc as plsc

MAXSEG = 256
L = 16


def _build_sc_scatter(B, C, HW):
    NW = 32
    tecs_per_b = NW // B
    pix_per_tec = HW // tecs_per_b
    P = 512
    nchunk = pix_per_tec // P
    groups = P // L
    mesh = plsc.VectorSubcoreMesh(core_axis_name="c", subcore_axis_name="s",
                                  num_cores=2, num_subcores=16)

    @functools.partial(
        pl.kernel,
        out_type=[
            jax.ShapeDtypeStruct((NW, MAXSEG * C), jnp.float32),
            jax.ShapeDtypeStruct((NW, MAXSEG), jnp.float32),
        ],
        mesh=mesh,
        compiler_params=pltpu.CompilerParams(needs_layout_passes=False),
        scratch_types=[
            pltpu.VMEM((C, P), jnp.float32),
            pltpu.VMEM((P,), jnp.int32),
            pltpu.VMEM((MAXSEG * C,), jnp.float32),
            pltpu.VMEM((L * MAXSEG,), jnp.float32),
            pltpu.VMEM((MAXSEG,), jnp.float32),
        ],
    )
    def body(feat, seg, psums, pcounts, fbuf, sbuf, acc, hist, cbuf):
        wid = lax.axis_index("s") * 2 + lax.axis_index("c")
        b = wid // tecs_per_b
        slot = wid % tecs_per_b
        base_pix = slot * pix_per_tec

        zf = jnp.zeros((L,), jnp.float32)
        onesf = jnp.ones((L,), jnp.float32)
        lane = lax.iota(jnp.int32, L)
        hist_off = lane * MAXSEG

        def zacc(i, _):
            acc[pl.ds(i * L, L)] = zf
            return 0

        lax.fori_loop(0, MAXSEG * C // L, zacc, 0)

        def zhist(i, _):
            hist[pl.ds(i * L, L)] = zf
            return 0

        lax.fori_loop(0, MAXSEG, zhist, 0)

        def chunk_body(k, _):
            p0 = base_pix + k * P
            pltpu.sync_copy(seg.at[b, pl.ds(p0, P)], sbuf)
            pltpu.sync_copy(feat.at[b, :, pl.ds(p0, P)], fbuf)

            def group_body(g, _):
                off = g * L
                segv = sbuf[pl.ds(off, L)]
                plsc.addupdate_scatter(hist, [segv + hist_off], onesf)
                segc = segv * C
                for c in range(C):
                    vals = fbuf[c, pl.ds(off, L)]
                    plsc.addupdate_scatter(acc, [segc + c], vals)
                return 0

            lax.fori_loop(0, groups, group_body, 0)
            return 0

        lax.fori_loop(0, nchunk, chunk_body, 0)

        for sb in range(MAXSEG // L):
            v = hist[pl.ds(sb * L, L)]
            for ln in range(1, L):
                v = v + hist[pl.ds(ln * MAXSEG + sb * L, L)]
            cbuf[pl.ds(sb * L, L)] = v

        pltpu.sync_copy(acc, psums.at[wid])
        pltpu.sync_copy(cbuf, pcounts.at[wid])

    return body


def _combine_body(ps_ref, pc_ref, cnx_ref, cny_ref, wx_ref, wy_ref, bp_ref,
                  inv_wh_ref, out_ref):
    sums = jnp.sum(ps_ref[...], axis=1)
    counts = jnp.sum(pc_ref[...], axis=1)
    emb = sums / jnp.clip(counts, 1.0, None)[..., None]
    inv_w = inv_wh_ref[0, 0]
    inv_h = inv_wh_ref[0, 1]
    pos = (cnx_ref[...][..., None] * inv_w * wx_ref[...]
           + cny_ref[...][..., None] * inv_h * wy_ref[...]
           + bp_ref[...])
    out_ref[...] = emb + pos


def kernel(img, features, segments, centroid_coords, W_pos, b_pos):
    B, C, H, W = features.shape
    _, _, height, width = img.shape
    HW = H * W
    feat = features.reshape(B, C, HW)
    seg = segments.reshape(B, HW).astype(jnp.int32)

    psums, pcounts = _build_sc_scatter(B, C, HW)(feat, seg)
    ps4 = psums.reshape(B, 32 // B, MAXSEG, C)
    pc4 = pcounts.reshape(B, 32 // B, MAXSEG)

    cnx = centroid_coords[..., 0]
    cny = centroid_coords[..., 1]
    wx = W_pos[:, 0].reshape(1, 1, C)
    wy = W_pos[:, 1].reshape(1, 1, C)
    bp = b_pos.reshape(1, 1, C)
    inv_wh = jnp.array([[1.0 / float(width), 1.0 / float(height)]],
                       dtype=jnp.float32)

    out = pl.pallas_call(
        _combine_body,
        out_shape=jax.ShapeDtypeStruct((B, MAXSEG, C), jnp.float32),
    )(ps4, pc4, cnx, cny, wx, wy, bp, inv_wh)
    return out


# speedup vs baseline: 1.6820x; 1.1767x over previous
import functools

import jax
import jax.numpy as jnp
from jax import lax
from jax.experimental import pallas as pl
from jax.experimental.pallas import tpu as pltpu
from jax.experimental.pallas import tpu_sc as plsc

MAXSEG = 256
L = 16


def _build_sc_scatter(B, C, HW):
    NW = 32
    tecs_per_b = NW // B
    pix_per_tec = HW // tecs_per_b
    P = 512
    nchunk = pix_per_tec // P
    groups = P // L
    mesh = plsc.VectorSubcoreMesh(core_axis_name="c", subcore_axis_name="s",
                                  num_cores=2, num_subcores=16)

    @functools.partial(
        pl.kernel,
        out_type=[
            jax.ShapeDtypeStruct((NW, MAXSEG * C), jnp.float32),
            jax.ShapeDtypeStruct((NW, MAXSEG), jnp.float32),
        ],
        mesh=mesh,
        compiler_params=pltpu.CompilerParams(needs_layout_passes=False),
        scratch_types=[
            pltpu.VMEM((C, P), jnp.float32),
            pltpu.VMEM((P,), jnp.int32),
            pltpu.VMEM((MAXSEG * C,), jnp.float32),
            pltpu.VMEM((L * MAXSEG,), jnp.float32),
            pltpu.VMEM((MAXSEG,), jnp.float32),
        ],
    )
    def body(feat, seg, psums, pcounts, fbuf, sbuf, acc, hist, cbuf):
        wid = lax.axis_index("s") * 2 + lax.axis_index("c")
        b = wid // tecs_per_b
        slot = wid % tecs_per_b
        base_pix = slot * pix_per_tec

        zf = jnp.zeros((L,), jnp.float32)
        onesf = jnp.ones((L,), jnp.float32)
        lane = lax.iota(jnp.int32, L)
        hist_off = lane * MAXSEG

        @plsc.parallel_loop(0, MAXSEG * C // L, unroll=8)
        def zacc(i):
            acc[pl.ds(i * L, L)] = zf

        @plsc.parallel_loop(0, MAXSEG, unroll=8)
        def zhist(i):
            hist[pl.ds(i * L, L)] = zf

        def chunk_body(k, _):
            p0 = base_pix + k * P
            pltpu.sync_copy(seg.at[b, pl.ds(p0, P)], sbuf)
            pltpu.sync_copy(feat.at[b, :, pl.ds(p0, P)], fbuf)

            @plsc.parallel_loop(0, groups, unroll=2)
            def group_body(g):
                off = g * L
                segv = sbuf[pl.ds(off, L)]
                plsc.addupdate_scatter(hist, [segv + hist_off], onesf)
                segc = segv * C
                for c in range(C):
                    vals = fbuf[c, pl.ds(off, L)]
                    plsc.addupdate_scatter(acc, [segc + c], vals)

            return 0

        lax.fori_loop(0, nchunk, chunk_body, 0)

        for sb in range(MAXSEG // L):
            v = hist[pl.ds(sb * L, L)]
            for ln in range(1, L):
                v = v + hist[pl.ds(ln * MAXSEG + sb * L, L)]
            cbuf[pl.ds(sb * L, L)] = v

        pltpu.sync_copy(acc, psums.at[wid])
        pltpu.sync_copy(cbuf, pcounts.at[wid])

    return body


def _combine_body(ps_ref, pc_ref, cnx_ref, cny_ref, wx_ref, wy_ref, bp_ref,
                  inv_wh_ref, out_ref):
    sums = jnp.sum(ps_ref[...], axis=1)
    counts = jnp.sum(pc_ref[...], axis=1)
    emb = sums / jnp.clip(counts, 1.0, None)[..., None]
    inv_w = inv_wh_ref[0, 0]
    inv_h = inv_wh_ref[0, 1]
    pos = (cnx_ref[...][..., None] * inv_w * wx_ref[...]
           + cny_ref[...][..., None] * inv_h * wy_ref[...]
           + bp_ref[...])
    out_ref[...] = emb + pos


def kernel(img, features, segments, centroid_coords, W_pos, b_pos):
    B, C, H, W = features.shape
    _, _, height, width = img.shape
    HW = H * W
    feat = features.reshape(B, C, HW)
    seg = segments.reshape(B, HW).astype(jnp.int32)

    psums, pcounts = _build_sc_scatter(B, C, HW)(feat, seg)
    ps4 = psums.reshape(B, 32 // B, MAXSEG, C)
    pc4 = pcounts.reshape(B, 32 // B, MAXSEG)

    cnx = centroid_coords[..., 0]
    cny = centroid_coords[..., 1]
    wx = W_pos[:, 0].reshape(1, 1, C)
    wy = W_pos[:, 1].reshape(1, 1, C)
    bp = b_pos.reshape(1, 1, C)
    inv_wh = jnp.array([[1.0 / float(width), 1.0 / float(height)]],
                       dtype=jnp.float32)

    out = pl.pallas_call(
        _combine_body,
        out_shape=jax.ShapeDtypeStruct((B, MAXSEG, C), jnp.float32),
    )(ps4, pc4, cnx, cny, wx, wy, bp, inv_wh)
    return out


# speedup vs baseline: 3.8763x; 2.3046x over previous
import functools

import jax
import jax.numpy as jnp
from jax import lax
from jax.experimental import pallas as pl
from jax.experimental.pallas import tpu as pltpu
from jax.experimental.pallas import tpu_sc as plsc

MAXSEG = 256
L = 16
CPAD = 97


def _build_sc_scatter(B, C, HW):
    NW = 32
    tecs_per_b = NW // B
    pix_per_tec = HW // tecs_per_b
    P = 512
    nchunk = pix_per_tec // P
    groups = P // L
    mesh = plsc.VectorSubcoreMesh(core_axis_name="c", subcore_axis_name="s",
                                  num_cores=2, num_subcores=16)

    @functools.partial(
        pl.kernel,
        out_type=[
            jax.ShapeDtypeStruct((NW, MAXSEG * CPAD), jnp.float32),
            jax.ShapeDtypeStruct((NW, MAXSEG), jnp.float32),
        ],
        mesh=mesh,
        compiler_params=pltpu.CompilerParams(needs_layout_passes=False),
        scratch_types=[
            pltpu.VMEM((C, P), jnp.float32),
            pltpu.VMEM((P,), jnp.int32),
            pltpu.VMEM((MAXSEG * CPAD,), jnp.float32),
            pltpu.VMEM((L * MAXSEG,), jnp.float32),
            pltpu.VMEM((MAXSEG,), jnp.float32),
        ],
    )
    def body(feat, seg, psums, pcounts, fbuf, sbuf, acc, hist, cbuf):
        wid = lax.axis_index("s") * 2 + lax.axis_index("c")
        b = wid // tecs_per_b
        slot = wid % tecs_per_b
        base_pix = slot * pix_per_tec

        zf = jnp.zeros((L,), jnp.float32)
        onesf = jnp.ones((L,), jnp.float32)
        lane = lax.iota(jnp.int32, L)
        hist_off = lane * MAXSEG

        @plsc.parallel_loop(0, MAXSEG * CPAD // L, unroll=8)
        def zacc(i):
            acc[pl.ds(i * L, L)] = zf

        @plsc.parallel_loop(0, MAXSEG, unroll=8)
        def zhist(i):
            hist[pl.ds(i * L, L)] = zf

        def chunk_body(k, _):
            p0 = base_pix + k * P
            pltpu.sync_copy(seg.at[b, pl.ds(p0, P)], sbuf)
            pltpu.sync_copy(feat.at[b, :, pl.ds(p0, P)], fbuf)

            @plsc.parallel_loop(0, groups, unroll=2)
            def group_body(g):
                off = g * L
                segv = sbuf[pl.ds(off, L)]
                plsc.addupdate_scatter(hist, [segv + hist_off], onesf)
                segc = segv * CPAD
                for c in range(C):
                    vals = fbuf[c, pl.ds(off, L)]
                    plsc.addupdate_scatter(acc, [segc + c], vals)

            return 0

        lax.fori_loop(0, nchunk, chunk_body, 0)

        for sb in range(MAXSEG // L):
            v = hist[pl.ds(sb * L, L)]
            for ln in range(1, L):
                v = v + hist[pl.ds(ln * MAXSEG + sb * L, L)]
            cbuf[pl.ds(sb * L, L)] = v

        pltpu.sync_copy(acc, psums.at[wid])
        pltpu.sync_copy(cbuf, pcounts.at[wid])

    return body


def _combine_body(ps_ref, pc_ref, cnx_ref, cny_ref, wx_ref, wy_ref, bp_ref,
                  inv_wh_ref, out_ref):
    sums = jnp.sum(ps_ref[...], axis=1)[..., :96]
    counts = jnp.sum(pc_ref[...], axis=1)
    emb = sums / jnp.clip(counts, 1.0, None)[..., None]
    inv_w = inv_wh_ref[0, 0]
    inv_h = inv_wh_ref[0, 1]
    pos = (cnx_ref[...][..., None] * inv_w * wx_ref[...]
           + cny_ref[...][..., None] * inv_h * wy_ref[...]
           + bp_ref[...])
    out_ref[...] = emb + pos


def kernel(img, features, segments, centroid_coords, W_pos, b_pos):
    B, C, H, W = features.shape
    _, _, height, width = img.shape
    HW = H * W
    feat = features.reshape(B, C, HW)
    seg = segments.reshape(B, HW).astype(jnp.int32)

    psums, pcounts = _build_sc_scatter(B, C, HW)(feat, seg)
    ps4 = psums.reshape(B, 32 // B, MAXSEG, CPAD)
    pc4 = pcounts.reshape(B, 32 // B, MAXSEG)

    cnx = centroid_coords[..., 0]
    cny = centroid_coords[..., 1]
    wx = W_pos[:, 0].reshape(1, 1, C)
    wy = W_pos[:, 1].reshape(1, 1, C)
    bp = b_pos.reshape(1, 1, C)
    inv_wh = jnp.array([[1.0 / float(width), 1.0 / float(height)]],
                       dtype=jnp.float32)

    out = pl.pallas_call(
        _combine_body,
        out_shape=jax.ShapeDtypeStruct((B, MAXSEG, C), jnp.float32),
    )(ps4, pc4, cnx, cny, wx, wy, bp, inv_wh)
    return out


# speedup vs baseline: 3.9143x; 1.0098x over previous
import functools

import jax
import jax.numpy as jnp
from jax import lax
from jax.experimental import pallas as pl
from jax.experimental.pallas import tpu as pltpu
from jax.experimental.pallas import tpu_sc as plsc

MAXSEG = 256
L = 16
CPAD = 97


def _build_sc_scatter(B, C, HW):
    NW = 32
    tecs_per_b = NW // B
    pix_per_tec = HW // tecs_per_b
    P = 384
    nchunk = pix_per_tec // P
    groups = P // L
    mesh = plsc.VectorSubcoreMesh(core_axis_name="c", subcore_axis_name="s",
                                  num_cores=2, num_subcores=16)

    @functools.partial(
        pl.kernel,
        out_type=[
            jax.ShapeDtypeStruct((NW, MAXSEG * CPAD), jnp.float32),
            jax.ShapeDtypeStruct((NW, MAXSEG), jnp.float32),
        ],
        mesh=mesh,
        compiler_params=pltpu.CompilerParams(needs_layout_passes=False),
        scratch_types=[
            pltpu.VMEM((2, C, P), jnp.float32),
            pltpu.VMEM((2, P), jnp.int32),
            pltpu.VMEM((MAXSEG * CPAD,), jnp.float32),
            pltpu.VMEM((L * MAXSEG,), jnp.float32),
            pltpu.VMEM((MAXSEG,), jnp.float32),
            pltpu.SemaphoreType.DMA((2,)),
            pltpu.SemaphoreType.DMA((2,)),
        ],
    )
    def body(feat, seg, psums, pcounts, fbuf, sbuf, acc, hist, cbuf,
             sem_f, sem_s):
        wid = lax.axis_index("s") * 2 + lax.axis_index("c")
        b = wid // tecs_per_b
        slot = wid % tecs_per_b
        base_pix = slot * pix_per_tec

        zf = jnp.zeros((L,), jnp.float32)
        onesf = jnp.ones((L,), jnp.float32)
        lane = lax.iota(jnp.int32, L)
        hist_off = lane * MAXSEG

        @plsc.parallel_loop(0, MAXSEG * CPAD // L, unroll=8)
        def zacc(i):
            acc[pl.ds(i * L, L)] = zf

        @plsc.parallel_loop(0, MAXSEG, unroll=8)
        def zhist(i):
            hist[pl.ds(i * L, L)] = zf

        def issue(k, s):
            p0 = base_pix + k * P
            pltpu.async_copy(seg.at[b, pl.ds(p0, P)], sbuf.at[s], sem_s.at[s])
            pltpu.async_copy(feat.at[b, :, pl.ds(p0, P)], fbuf.at[s],
                             sem_f.at[s])

        def drain(s):
            pltpu.make_async_copy(seg.at[b, pl.ds(base_pix, P)], sbuf.at[s],
                                  sem_s.at[s]).wait()
            pltpu.make_async_copy(feat.at[b, :, pl.ds(base_pix, P)],
                                  fbuf.at[s], sem_f.at[s]).wait()

        issue(0, 0)
        issue(1, 1)

        def pair_body(kk, _):
            for s in range(2):
                k = kk * 2 + s
                drain(s)

                @plsc.parallel_loop(0, groups, unroll=2)
                def group_body(g):
                    off = g * L
                    segv = sbuf[s, pl.ds(off, L)]
                    plsc.addupdate_scatter(hist, [segv + hist_off], onesf)
                    segc = segv * CPAD
                    for c in range(C):
                        vals = fbuf[s, c, pl.ds(off, L)]
                        plsc.addupdate_scatter(acc, [segc + c], vals)

                @pl.when(k + 2 < nchunk)
                def _():
                    issue(k + 2, s)

            return 0

        lax.fori_loop(0, nchunk // 2, pair_body, 0)

        for sb in range(MAXSEG // L):
            v = hist[pl.ds(sb * L, L)]
            for ln in range(1, L):
                v = v + hist[pl.ds(ln * MAXSEG + sb * L, L)]
            cbuf[pl.ds(sb * L, L)] = v

        pltpu.sync_copy(acc, psums.at[wid])
        pltpu.sync_copy(cbuf, pcounts.at[wid])

    return body


def _combine_body(ps_ref, pc_ref, cnx_ref, cny_ref, wx_ref, wy_ref, bp_ref,
                  inv_wh_ref, out_ref):
    sums = jnp.sum(ps_ref[...], axis=1)[..., :96]
    counts = jnp.sum(pc_ref[...], axis=1)
    emb = sums / jnp.clip(counts, 1.0, None)[..., None]
    inv_w = inv_wh_ref[0, 0]
    inv_h = inv_wh_ref[0, 1]
    pos = (cnx_ref[...][..., None] * inv_w * wx_ref[...]
           + cny_ref[...][..., None] * inv_h * wy_ref[...]
           + bp_ref[...])
    out_ref[...] = emb + pos


def kernel(img, features, segments, centroid_coords, W_pos, b_pos):
    B, C, H, W = features.shape
    _, _, height, width = img.shape
    HW = H * W
    feat = features.reshape(B, C, HW)
    seg = segments.reshape(B, HW).astype(jnp.int32)

    psums, pcounts = _build_sc_scatter(B, C, HW)(feat, seg)
    ps4 = psums.reshape(B, 32 // B, MAXSEG, CPAD)
    pc4 = pcounts.reshape(B, 32 // B, MAXSEG)

    cnx = centroid_coords[..., 0]
    cny = centroid_coords[..., 1]
    wx = W_pos[:, 0].reshape(1, 1, C)
    wy = W_pos[:, 1].reshape(1, 1, C)
    bp = b_pos.reshape(1, 1, C)
    inv_wh = jnp.array([[1.0 / float(width), 1.0 / float(height)]],
                       dtype=jnp.float32)

    out = pl.pallas_call(
        _combine_body,
        out_shape=jax.ShapeDtypeStruct((B, MAXSEG, C), jnp.float32),
    )(ps4, pc4, cnx, cny, wx, wy, bp, inv_wh)
    return out


# speedup vs baseline: 6.9050x; 1.7641x over previous
import functools

import jax
import jax.numpy as jnp
from jax import lax
from jax.experimental import pallas as pl
from jax.experimental.pallas import tpu as pltpu
from jax.experimental.pallas import tpu_sc as plsc

MAXSEG = 256
L = 16
CPAD = 97


def _build_sc_scatter(B, C, H, W):
    NW = 32
    tecs_per_b = NW // B
    rows_per_tec = H // tecs_per_b
    P = W
    nchunk = rows_per_tec
    groups = P // L
    mesh = plsc.VectorSubcoreMesh(core_axis_name="c", subcore_axis_name="s",
                                  num_cores=2, num_subcores=16)

    @functools.partial(
        pl.kernel,
        out_type=[
            jax.ShapeDtypeStruct((NW, MAXSEG * CPAD), jnp.float32),
            jax.ShapeDtypeStruct((NW, MAXSEG), jnp.float32),
        ],
        mesh=mesh,
        compiler_params=pltpu.CompilerParams(needs_layout_passes=False),
        scratch_types=[
            pltpu.VMEM((2, C, 1, P), jnp.float32),
            pltpu.VMEM((2, 1, P), jnp.int32),
            pltpu.VMEM((MAXSEG * CPAD,), jnp.float32),
            pltpu.VMEM((L * MAXSEG,), jnp.float32),
            pltpu.VMEM((MAXSEG,), jnp.float32),
            pltpu.SemaphoreType.DMA((2,)),
            pltpu.SemaphoreType.DMA((2,)),
        ],
    )
    def body(feat, seg, psums, pcounts, fbuf, sbuf, acc, hist, cbuf,
             sem_f, sem_s):
        wid = lax.axis_index("s") * 2 + lax.axis_index("c")
        b = wid // tecs_per_b
        slot = wid % tecs_per_b
        base_row = slot * rows_per_tec

        zf = jnp.zeros((L,), jnp.float32)
        onesf = jnp.ones((L,), jnp.float32)
        lane = lax.iota(jnp.int32, L)
        hist_off = lane * MAXSEG

        @plsc.parallel_loop(0, MAXSEG * CPAD // L, unroll=8)
        def zacc(i):
            acc[pl.ds(i * L, L)] = zf

        @plsc.parallel_loop(0, MAXSEG, unroll=8)
        def zhist(i):
            hist[pl.ds(i * L, L)] = zf

        def issue(k, s):
            r = base_row + k
            pltpu.async_copy(seg.at[b, pl.ds(r, 1), :], sbuf.at[s],
                             sem_s.at[s])
            pltpu.async_copy(feat.at[b, :, pl.ds(r, 1), :], fbuf.at[s],
                             sem_f.at[s])

        def drain(s):
            pltpu.make_async_copy(seg.at[b, pl.ds(base_row, 1), :],
                                  sbuf.at[s], sem_s.at[s]).wait()
            pltpu.make_async_copy(feat.at[b, :, pl.ds(base_row, 1), :],
                                  fbuf.at[s], sem_f.at[s]).wait()

        issue(0, 0)
        issue(1, 1)

        def pair_body(kk, _):
            for s in range(2):
                k = kk * 2 + s
                drain(s)

                @plsc.parallel_loop(0, groups, unroll=2)
                def group_body(g):
                    off = g * L
                    segv = sbuf[s, 0, pl.ds(off, L)]
                    plsc.addupdate_scatter(hist, [segv + hist_off], onesf)
                    segc = segv * CPAD
                    for c in range(C):
                        vals = fbuf[s, c, 0, pl.ds(off, L)]
                        plsc.addupdate_scatter(acc, [segc + c], vals)

                @pl.when(k + 2 < nchunk)
                def _():
                    issue(k + 2, s)

            return 0

        lax.fori_loop(0, nchunk // 2, pair_body, 0)

        for sb in range(MAXSEG // L):
            v = hist[pl.ds(sb * L, L)]
            for ln in range(1, L):
                v = v + hist[pl.ds(ln * MAXSEG + sb * L, L)]
            cbuf[pl.ds(sb * L, L)] = v

        pltpu.sync_copy(acc, psums.at[wid])
        pltpu.sync_copy(cbuf, pcounts.at[wid])

    return body


def _combine_body(ps_ref, pc_ref, cnx_ref, cny_ref, wx_ref, wy_ref, bp_ref,
                  inv_wh_ref, out_ref):
    sums = jnp.sum(ps_ref[...], axis=1)[..., :96]
    counts = jnp.sum(pc_ref[...], axis=1)
    emb = sums / jnp.clip(counts, 1.0, None)[..., None]
    inv_w = inv_wh_ref[0, 0]
    inv_h = inv_wh_ref[0, 1]
    pos = (cnx_ref[...][..., None] * inv_w * wx_ref[...]
           + cny_ref[...][..., None] * inv_h * wy_ref[...]
           + bp_ref[...])
    out_ref[...] = emb + pos


def kernel(img, features, segments, centroid_coords, W_pos, b_pos):
    B, C, H, W = features.shape
    _, _, height, width = img.shape
    seg = segments.astype(jnp.int32)

    psums, pcounts = _build_sc_scatter(B, C, H, W)(features, seg)
    ps4 = psums.reshape(B, 32 // B, MAXSEG, CPAD)
    pc4 = pcounts.reshape(B, 32 // B, MAXSEG)

    cnx = centroid_coords[..., 0]
    cny = centroid_coords[..., 1]
    wx = W_pos[:, 0].reshape(1, 1, C)
    wy = W_pos[:, 1].reshape(1, 1, C)
    bp = b_pos.reshape(1, 1, C)
    inv_wh = jnp.array([[1.0 / float(width), 1.0 / float(height)]],
                       dtype=jnp.float32)

    out = pl.pallas_call(
        _combine_body,
        out_shape=jax.ShapeDtypeStruct((B, MAXSEG, C), jnp.float32),
    )(ps4, pc4, cnx, cny, wx, wy, bp, inv_wh)
    return out


# speedup vs baseline: 7.6482x; 1.1076x over previous
import functools

import jax
import jax.numpy as jnp
from jax import lax
from jax.experimental import pallas as pl
from jax.experimental.pallas import tpu as pltpu
from jax.experimental.pallas import tpu_sc as plsc

MAXSEG = 256
L = 16
CPAD = 97


def _build_sc_scatter(B, C, H, W):
    NW = 32
    tecs_per_b = NW // B
    rows_per_tec = H // tecs_per_b
    P = W
    nchunk = rows_per_tec
    groups = P // L
    mesh = plsc.VectorSubcoreMesh(core_axis_name="c", subcore_axis_name="s",
                                  num_cores=2, num_subcores=16)

    @functools.partial(
        pl.kernel,
        out_type=[
            jax.ShapeDtypeStruct((NW, MAXSEG * CPAD), jnp.float32),
            jax.ShapeDtypeStruct((NW, MAXSEG), jnp.float32),
        ],
        mesh=mesh,
        compiler_params=pltpu.CompilerParams(needs_layout_passes=False),
        scratch_types=[
            pltpu.VMEM((2, C, 1, P), jnp.float32),
            pltpu.VMEM((2, 1, P), jnp.int32),
            pltpu.VMEM((MAXSEG * CPAD,), jnp.float32),
            pltpu.VMEM((L * MAXSEG,), jnp.float32),
            pltpu.VMEM((MAXSEG,), jnp.float32),
            pltpu.VMEM((L * CPAD,), jnp.float32),
            pltpu.SemaphoreType.DMA((2,)),
            pltpu.SemaphoreType.DMA((2,)),
        ],
    )
    def body(feat, seg, psums, pcounts, fbuf, sbuf, acc, hist, cbuf, tbuf,
             sem_f, sem_s):
        wid = lax.axis_index("s") * 2 + lax.axis_index("c")
        b = wid // tecs_per_b
        slot = wid % tecs_per_b
        base_row = slot * rows_per_tec

        zf = jnp.zeros((L,), jnp.float32)
        onesf = jnp.ones((L,), jnp.float32)
        lane = lax.iota(jnp.int32, L)
        hist_off = lane * MAXSEG

        @plsc.parallel_loop(0, MAXSEG * CPAD // L, unroll=8)
        def zacc(i):
            acc[pl.ds(i * L, L)] = zf

        @plsc.parallel_loop(0, MAXSEG, unroll=8)
        def zhist(i):
            hist[pl.ds(i * L, L)] = zf

        def issue(k, s):
            r = base_row + k
            pltpu.async_copy(seg.at[b, pl.ds(r, 1), :], sbuf.at[s],
                             sem_s.at[s])
            pltpu.async_copy(feat.at[b, :, pl.ds(r, 1), :], fbuf.at[s],
                             sem_f.at[s])

        def drain(s):
            pltpu.make_async_copy(seg.at[b, pl.ds(base_row, 1), :],
                                  sbuf.at[s], sem_s.at[s]).wait()
            pltpu.make_async_copy(feat.at[b, :, pl.ds(base_row, 1), :],
                                  fbuf.at[s], sem_f.at[s]).wait()

        issue(0, 0)
        issue(1, 1)

        def pair_body(kk, _):
            for s in range(2):
                k = kk * 2 + s
                drain(s)

                @plsc.parallel_loop(0, groups, unroll=2)
                def group_body(g):
                    off = g * L
                    segv = sbuf[s, 0, pl.ds(off, L)]
                    plsc.addupdate_scatter(hist, [segv + hist_off], onesf)
                    segc = segv * CPAD
                    segs = [segc + r for r in range(8)]
                    for c in range(C):
                        vals = fbuf[s, c, 0, pl.ds(off, L)]
                        plsc.addupdate_scatter(
                            acc.at[pl.ds((c // 8) * 8, MAXSEG * CPAD - 88)],
                            [segs[c % 8]], vals)

                @pl.when(k + 2 < nchunk)
                def _():
                    issue(k + 2, s)

            return 0

        lax.fori_loop(0, nchunk // 2, pair_body, 0)

        for sb in range(MAXSEG // L):
            v = hist[pl.ds(sb * L, L)]
            for ln in range(1, L):
                v = v + hist[pl.ds(ln * MAXSEG + sb * L, L)]
            cbuf[pl.ds(sb * L, L)] = v

        pltpu.sync_copy(acc, psums.at[wid])
        pltpu.sync_copy(cbuf, pcounts.at[wid])

    return body


def _combine_body(ps_ref, pc_ref, cnx_ref, cny_ref, wx_ref, wy_ref, bp_ref,
                  inv_wh_ref, out_ref):
    sums = jnp.sum(ps_ref[...], axis=1)[..., :96]
    counts = jnp.sum(pc_ref[...], axis=1)
    emb = sums / jnp.clip(counts, 1.0, None)[..., None]
    inv_w = inv_wh_ref[0, 0]
    inv_h = inv_wh_ref[0, 1]
    pos = (cnx_ref[...][..., None] * inv_w * wx_ref[...]
           + cny_ref[...][..., None] * inv_h * wy_ref[...]
           + bp_ref[...])
    out_ref[...] = emb + pos


def kernel(img, features, segments, centroid_coords, W_pos, b_pos):
    B, C, H, W = features.shape
    _, _, height, width = img.shape
    seg = segments.astype(jnp.int32)

    psums, pcounts = _build_sc_scatter(B, C, H, W)(features, seg)
    ps4 = psums.reshape(B, 32 // B, MAXSEG, CPAD)
    pc4 = pcounts.reshape(B, 32 // B, MAXSEG)

    cnx = centroid_coords[..., 0]
    cny = centroid_coords[..., 1]
    wx = W_pos[:, 0].reshape(1, 1, C)
    wy = W_pos[:, 1].reshape(1, 1, C)
    bp = b_pos.reshape(1, 1, C)
    inv_wh = jnp.array([[1.0 / float(width), 1.0 / float(height)]],
                       dtype=jnp.float32)

    out = pl.pallas_call(
        _combine_body,
        out_shape=jax.ShapeDtypeStruct((B, MAXSEG, C), jnp.float32),
    )(ps4, pc4, cnx, cny, wx, wy, bp, inv_wh)
    return out


# speedup vs baseline: 9.6566x; 1.2626x over previous
import functools

import jax
import jax.numpy as jnp
from jax import lax
from jax.experimental import pallas as pl
from jax.experimental.pallas import tpu as pltpu
from jax.experimental.pallas import tpu_sc as plsc

MAXSEG = 256
L = 16
CPAD = 97
H_SC = 288
ROWS_T = 8


def _build_sc_scatter(B, C, W):
    NW = 32
    tecs_per_b = NW // B
    rows_per_tec = H_SC // tecs_per_b
    P = W
    nchunk = rows_per_tec
    groups = P // L
    mesh = plsc.VectorSubcoreMesh(core_axis_name="c", subcore_axis_name="s",
                                  num_cores=2, num_subcores=16)

    @functools.partial(
        pl.kernel,
        out_type=[
            jax.ShapeDtypeStruct((NW, MAXSEG * CPAD), jnp.float32),
            jax.ShapeDtypeStruct((NW, MAXSEG), jnp.float32),
        ],
        mesh=mesh,
        compiler_params=pltpu.CompilerParams(needs_layout_passes=False),
        scratch_types=[
            pltpu.VMEM((2, C, 1, P), jnp.float32),
            pltpu.VMEM((2, 1, P), jnp.int32),
            pltpu.VMEM((MAXSEG * CPAD,), jnp.float32),
            pltpu.VMEM((L * MAXSEG,), jnp.float32),
            pltpu.VMEM((MAXSEG,), jnp.float32),
            pltpu.SemaphoreType.DMA((2,)),
            pltpu.SemaphoreType.DMA((2,)),
        ],
    )
    def body(feat, seg, psums, pcounts, fbuf, sbuf, acc, hist, cbuf,
             sem_f, sem_s):
        wid = lax.axis_index("s") * 2 + lax.axis_index("c")
        b = wid // tecs_per_b
        slot = wid % tecs_per_b
        base_row = slot * rows_per_tec

        zf = jnp.zeros((L,), jnp.float32)
        onesf = jnp.ones((L,), jnp.float32)
        lane = lax.iota(jnp.int32, L)
        hist_off = lane * MAXSEG

        @plsc.parallel_loop(0, MAXSEG * CPAD // L, unroll=8)
        def zacc(i):
            acc[pl.ds(i * L, L)] = zf

        @plsc.parallel_loop(0, MAXSEG, unroll=8)
        def zhist(i):
            hist[pl.ds(i * L, L)] = zf

        def issue(k, s):
            r = base_row + k
            pltpu.async_copy(seg.at[b, pl.ds(r, 1), :], sbuf.at[s],
                             sem_s.at[s])
            pltpu.async_copy(feat.at[b, :, pl.ds(r, 1), :], fbuf.at[s],
                             sem_f.at[s])

        def drain(s):
            pltpu.make_async_copy(seg.at[b, pl.ds(base_row, 1), :],
                                  sbuf.at[s], sem_s.at[s]).wait()
            pltpu.make_async_copy(feat.at[b, :, pl.ds(base_row, 1), :],
                                  fbuf.at[s], sem_f.at[s]).wait()

        issue(0, 0)
        issue(1, 1)

        def pair_body(kk, _):
            for s in range(2):
                k = kk * 2 + s
                drain(s)

                @plsc.parallel_loop(0, groups, unroll=2)
                def group_body(g):
                    off = g * L
                    segv = sbuf[s, 0, pl.ds(off, L)]
                    plsc.addupdate_scatter(hist, [segv + hist_off], onesf)
                    segc = segv * CPAD
                    segs = [segc + r for r in range(8)]
                    for c in range(C):
                        vals = fbuf[s, c, 0, pl.ds(off, L)]
                        plsc.addupdate_scatter(
                            acc.at[pl.ds((c // 8) * 8, MAXSEG * CPAD - 88)],
                            [segs[c % 8]], vals)

                @pl.when(k + 2 < nchunk)
                def _():
                    issue(k + 2, s)

            return 0

        lax.fori_loop(0, nchunk // 2, pair_body, 0)

        for sb in range(MAXSEG // L):
            v = hist[pl.ds(sb * L, L)]
            for ln in range(1, L):
                v = v + hist[pl.ds(ln * MAXSEG + sb * L, L)]
            cbuf[pl.ds(sb * L, L)] = v

        pltpu.sync_copy(acc, psums.at[wid])
        pltpu.sync_copy(cbuf, pcounts.at[wid])

    return body


def _tc_rows_body(seg_ref, feat_ref, sums_ref, cnt_ref):
    t = pl.program_id(1)
    n = ROWS_T * seg_ref.shape[2]
    seg = seg_ref[...].reshape(1, n)
    sids = lax.broadcasted_iota(jnp.int32, (MAXSEG, n), 0)
    oh = (sids == seg).astype(jnp.bfloat16)
    fb = feat_ref[...].reshape(feat_ref.shape[1], n).astype(jnp.bfloat16)
    part = lax.dot_general(oh, fb, (((1,), (1,)), ((), ())),
                           preferred_element_type=jnp.float32)
    ones_row = jnp.ones((1, n), jnp.bfloat16)
    cnt = lax.dot_general(ones_row, oh, (((1,), (1,)), ((), ())),
                          preferred_element_type=jnp.float32)

    @pl.when(t == 0)
    def _():
        sums_ref[0] = part
        cnt_ref[0] = cnt

    @pl.when(t != 0)
    def _():
        sums_ref[0] += part
        cnt_ref[0] += cnt


def _combine_body(ps_ref, pc_ref, ts_ref, tn_ref, cnx_ref, cny_ref, wx_ref,
                  wy_ref, bp_ref, inv_wh_ref, out_ref):
    sums = jnp.sum(ps_ref[...], axis=1)[..., :96] + ts_ref[...]
    counts = jnp.sum(pc_ref[...], axis=1) + tn_ref[..., 0, :]
    emb = sums / jnp.clip(counts, 1.0, None)[..., None]
    inv_w = inv_wh_ref[0, 0]
    inv_h = inv_wh_ref[0, 1]
    pos = (cnx_ref[...][..., None] * inv_w * wx_ref[...]
           + cny_ref[...][..., None] * inv_h * wy_ref[...]
           + bp_ref[...])
    out_ref[...] = emb + pos


def kernel(img, features, segments, centroid_coords, W_pos, b_pos):
    B, C, H, W = features.shape
    _, _, height, width = img.shape
    seg = segments.astype(jnp.int32)

    psums, pcounts = _build_sc_scatter(B, C, W)(features, seg)
    ps4 = psums.reshape(B, 32 // B, MAXSEG, CPAD)
    pc4 = pcounts.reshape(B, 32 // B, MAXSEG)

    nt = (H - H_SC) // ROWS_T
    row0 = H_SC // ROWS_T
    tc_sums, tc_cnt = pl.pallas_call(
        _tc_rows_body,
        grid=(B, nt),
        in_specs=[
            pl.BlockSpec((1, ROWS_T, W), lambda b, t: (b, row0 + t, 0)),
            pl.BlockSpec((1, C, ROWS_T, W), lambda b, t: (b, 0, row0 + t, 0)),
        ],
        out_specs=[
            pl.BlockSpec((1, MAXSEG, C), lambda b, t: (b, 0, 0)),
            pl.BlockSpec((1, 1, MAXSEG), lambda b, t: (b, 0, 0)),
        ],
        out_shape=[
            jax.ShapeDtypeStruct((B, MAXSEG, C), jnp.float32),
            jax.ShapeDtypeStruct((B, 1, MAXSEG), jnp.float32),
        ],
        compiler_params=pltpu.CompilerParams(
            dimension_semantics=("parallel", "arbitrary")),
    )(seg, features)

    cnx = centroid_coords[..., 0]
    cny = centroid_coords[..., 1]
    wx = W_pos[:, 0].reshape(1, 1, C)
    wy = W_pos[:, 1].reshape(1, 1, C)
    bp = b_pos.reshape(1, 1, C)
    inv_wh = jnp.array([[1.0 / float(width), 1.0 / float(height)]],
                       dtype=jnp.float32)

    out = pl.pallas_call(
        _combine_body,
        out_shape=jax.ShapeDtypeStruct((B, MAXSEG, C), jnp.float32),
    )(ps4, pc4, tc_sums, tc_cnt, cnx, cny, wx, wy, bp, inv_wh)
    return out


# speedup vs baseline: 11.7829x; 1.2202x over previous
import functools

import jax
import jax.numpy as jnp
from jax import lax
from jax.experimental import pallas as pl
from jax.experimental.pallas import tpu as pltpu
from jax.experimental.pallas import tpu_sc as plsc

MAXSEG = 256
L = 16
CPAD = 97
H_SC = 224
ROWS_T = 8


def _build_sc_scatter(B, C, W):
    NW = 32
    tecs_per_b = NW // B
    rows_per_tec = H_SC // tecs_per_b
    P = W
    nchunk = rows_per_tec
    groups = P // L
    mesh = plsc.VectorSubcoreMesh(core_axis_name="c", subcore_axis_name="s",
                                  num_cores=2, num_subcores=16)

    @functools.partial(
        pl.kernel,
        out_type=[
            jax.ShapeDtypeStruct((NW, MAXSEG * CPAD), jnp.float32),
            jax.ShapeDtypeStruct((NW, MAXSEG), jnp.float32),
        ],
        mesh=mesh,
        compiler_params=pltpu.CompilerParams(needs_layout_passes=False),
        scratch_types=[
            pltpu.VMEM((2, C, 1, P), jnp.float32),
            pltpu.VMEM((2, 1, P), jnp.int32),
            pltpu.VMEM((MAXSEG * CPAD,), jnp.float32),
            pltpu.VMEM((L * MAXSEG,), jnp.float32),
            pltpu.VMEM((MAXSEG,), jnp.float32),
            pltpu.SemaphoreType.DMA((2,)),
            pltpu.SemaphoreType.DMA((2,)),
        ],
    )
    def body(feat, seg, psums, pcounts, fbuf, sbuf, acc, hist, cbuf,
             sem_f, sem_s):
        wid = lax.axis_index("s") * 2 + lax.axis_index("c")
        b = wid // tecs_per_b
        slot = wid % tecs_per_b
        base_row = slot * rows_per_tec

        zf = jnp.zeros((L,), jnp.float32)
        onesf = jnp.ones((L,), jnp.float32)
        lane = lax.iota(jnp.int32, L)
        hist_off = lane * MAXSEG

        @plsc.parallel_loop(0, MAXSEG * CPAD // L, unroll=8)
        def zacc(i):
            acc[pl.ds(i * L, L)] = zf

        @plsc.parallel_loop(0, MAXSEG, unroll=8)
        def zhist(i):
            hist[pl.ds(i * L, L)] = zf

        def issue(k, s):
            r = base_row + k
            pltpu.async_copy(seg.at[b, pl.ds(r, 1), :], sbuf.at[s],
                             sem_s.at[s])
            pltpu.async_copy(feat.at[b, :, pl.ds(r, 1), :], fbuf.at[s],
                             sem_f.at[s])

        def drain(s):
            pltpu.make_async_copy(seg.at[b, pl.ds(base_row, 1), :],
                                  sbuf.at[s], sem_s.at[s]).wait()
            pltpu.make_async_copy(feat.at[b, :, pl.ds(base_row, 1), :],
                                  fbuf.at[s], sem_f.at[s]).wait()

        issue(0, 0)
        issue(1, 1)

        def pair_body(kk, _):
            for s in range(2):
                k = kk * 2 + s
                drain(s)

                @plsc.parallel_loop(0, groups, unroll=2)
                def group_body(g):
                    off = g * L
                    segv = sbuf[s, 0, pl.ds(off, L)]
                    plsc.addupdate_scatter(hist, [segv + hist_off], onesf)
                    segc = segv * CPAD
                    segs = [segc + r for r in range(8)]
                    for c in range(C):
                        vals = fbuf[s, c, 0, pl.ds(off, L)]
                        plsc.addupdate_scatter(
                            acc.at[pl.ds((c // 8) * 8, MAXSEG * CPAD - 88)],
                            [segs[c % 8]], vals)

                @pl.when(k + 2 < nchunk)
                def _():
                    issue(k + 2, s)

            return 0

        lax.fori_loop(0, nchunk // 2, pair_body, 0)

        for sb in range(MAXSEG // L):
            v = hist[pl.ds(sb * L, L)]
            for ln in range(1, L):
                v = v + hist[pl.ds(ln * MAXSEG + sb * L, L)]
            cbuf[pl.ds(sb * L, L)] = v

        pltpu.sync_copy(acc, psums.at[wid])
        pltpu.sync_copy(cbuf, pcounts.at[wid])

    return body


def _tc_rows_body(seg_ref, feat_ref, sums_ref, cnt_ref):
    t = pl.program_id(1)
    n = ROWS_T * seg_ref.shape[2]
    seg = seg_ref[...].reshape(1, n)
    sids = lax.broadcasted_iota(jnp.int32, (MAXSEG, n), 0)
    oh = (sids == seg).astype(jnp.bfloat16)
    fb = feat_ref[...].reshape(feat_ref.shape[1], n).astype(jnp.bfloat16)
    part = lax.dot_general(oh, fb, (((1,), (1,)), ((), ())),
                           preferred_element_type=jnp.float32)
    ones_row = jnp.ones((1, n), jnp.bfloat16)
    cnt = lax.dot_general(ones_row, oh, (((1,), (1,)), ((), ())),
                          preferred_element_type=jnp.float32)

    @pl.when(t == 0)
    def _():
        sums_ref[0] = part
        cnt_ref[0] = cnt

    @pl.when(t != 0)
    def _():
        sums_ref[0] += part
        cnt_ref[0] += cnt


def _combine_body(ps_ref, pc_ref, ts_ref, tn_ref, cnx_ref, cny_ref, wx_ref,
                  wy_ref, bp_ref, inv_wh_ref, out_ref):
    sums = jnp.sum(ps_ref[...], axis=1)[..., :96] + ts_ref[...]
    counts = jnp.sum(pc_ref[...], axis=1) + tn_ref[..., 0, :]
    emb = sums / jnp.clip(counts, 1.0, None)[..., None]
    inv_w = inv_wh_ref[0, 0]
    inv_h = inv_wh_ref[0, 1]
    pos = (cnx_ref[...][..., None] * inv_w * wx_ref[...]
           + cny_ref[...][..., None] * inv_h * wy_ref[...]
           + bp_ref[...])
    out_ref[...] = emb + pos


def kernel(img, features, segments, centroid_coords, W_pos, b_pos):
    B, C, H, W = features.shape
    _, _, height, width = img.shape
    seg = segments.astype(jnp.int32)

    psums, pcounts = _build_sc_scatter(B, C, W)(features, seg)
    ps4 = psums.reshape(B, 32 // B, MAXSEG, CPAD)
    pc4 = pcounts.reshape(B, 32 // B, MAXSEG)

    nt = (H - H_SC) // ROWS_T
    row0 = H_SC // ROWS_T
    tc_sums, tc_cnt = pl.pallas_call(
        _tc_rows_body,
        grid=(B, nt),
        in_specs=[
            pl.BlockSpec((1, ROWS_T, W), lambda b, t: (b, row0 + t, 0)),
            pl.BlockSpec((1, C, ROWS_T, W), lambda b, t: (b, 0, row0 + t, 0)),
        ],
        out_specs=[
            pl.BlockSpec((1, MAXSEG, C), lambda b, t: (b, 0, 0)),
            pl.BlockSpec((1, 1, MAXSEG), lambda b, t: (b, 0, 0)),
        ],
        out_shape=[
            jax.ShapeDtypeStruct((B, MAXSEG, C), jnp.float32),
            jax.ShapeDtypeStruct((B, 1, MAXSEG), jnp.float32),
        ],
        compiler_params=pltpu.CompilerParams(
            dimension_semantics=("parallel", "arbitrary")),
    )(seg, features)

    cnx = centroid_coords[..., 0]
    cny = centroid_coords[..., 1]
    wx = W_pos[:, 0].reshape(1, 1, C)
    wy = W_pos[:, 1].reshape(1, 1, C)
    bp = b_pos.reshape(1, 1, C)
    inv_wh = jnp.array([[1.0 / float(width), 1.0 / float(height)]],
                       dtype=jnp.float32)

    out = pl.pallas_call(
        _combine_body,
        out_shape=jax.ShapeDtypeStruct((B, MAXSEG, C), jnp.float32),
    )(ps4, pc4, tc_sums, tc_cnt, cnx, cny, wx, wy, bp, inv_wh)
    return out


# speedup vs baseline: 12.4200x; 1.0541x over previous
import functools

import jax
import jax.numpy as jnp
from jax import lax
from jax.experimental import pallas as pl
from jax.experimental.pallas import tpu as pltpu
from jax.experimental.pallas import tpu_sc as plsc

MAXSEG = 256
L = 16
CPAD = 97
H_SC = 208
ROWS_T = 8


def _build_sc_scatter(B, C, W):
    NW = 32
    tecs_per_b = NW // B
    rows_per_tec = H_SC // tecs_per_b
    P = W
    nchunk = rows_per_tec
    groups = P // L
    mesh = plsc.VectorSubcoreMesh(core_axis_name="c", subcore_axis_name="s",
                                  num_cores=2, num_subcores=16)

    @functools.partial(
        pl.kernel,
        out_type=[
            jax.ShapeDtypeStruct((NW, MAXSEG * CPAD), jnp.float32),
            jax.ShapeDtypeStruct((NW, MAXSEG), jnp.float32),
        ],
        mesh=mesh,
        compiler_params=pltpu.CompilerParams(needs_layout_passes=False),
        scratch_types=[
            pltpu.VMEM((2, C, 1, P), jnp.float32),
            pltpu.VMEM((2, 1, P), jnp.int32),
            pltpu.VMEM((MAXSEG * CPAD,), jnp.float32),
            pltpu.VMEM((L * MAXSEG,), jnp.float32),
            pltpu.VMEM((MAXSEG,), jnp.float32),
            pltpu.SemaphoreType.DMA((2,)),
            pltpu.SemaphoreType.DMA((2,)),
        ],
    )
    def body(feat, seg, psums, pcounts, fbuf, sbuf, acc, hist, cbuf,
             sem_f, sem_s):
        wid = lax.axis_index("s") * 2 + lax.axis_index("c")
        b = wid // tecs_per_b
        slot = wid % tecs_per_b
        base_row = slot * rows_per_tec

        zf = jnp.zeros((L,), jnp.float32)
        onesf = jnp.ones((L,), jnp.float32)
        lane = lax.iota(jnp.int32, L)
        hist_off = lane * MAXSEG

        @plsc.parallel_loop(0, MAXSEG * CPAD // L, unroll=8)
        def zacc(i):
            acc[pl.ds(i * L, L)] = zf

        @plsc.parallel_loop(0, MAXSEG, unroll=8)
        def zhist(i):
            hist[pl.ds(i * L, L)] = zf

        def issue(k, s):
            r = base_row + k
            pltpu.async_copy(seg.at[b, pl.ds(r, 1), :], sbuf.at[s],
                             sem_s.at[s])
            pltpu.async_copy(feat.at[b, :, pl.ds(r, 1), :], fbuf.at[s],
                             sem_f.at[s])

        def drain(s):
            pltpu.make_async_copy(seg.at[b, pl.ds(base_row, 1), :],
                                  sbuf.at[s], sem_s.at[s]).wait()
            pltpu.make_async_copy(feat.at[b, :, pl.ds(base_row, 1), :],
                                  fbuf.at[s], sem_f.at[s]).wait()

        issue(0, 0)
        issue(1, 1)

        def pair_body(kk, _):
            for s in range(2):
                k = kk * 2 + s
                drain(s)

                @plsc.parallel_loop(0, groups, unroll=2)
                def group_body(g):
                    off = g * L
                    segv = sbuf[s, 0, pl.ds(off, L)]
                    plsc.addupdate_scatter(hist, [segv + hist_off], onesf)
                    segc = segv * CPAD
                    segs = [segc + r for r in range(8)]
                    for c in range(C):
                        vals = fbuf[s, c, 0, pl.ds(off, L)]
                        plsc.addupdate_scatter(
                            acc.at[pl.ds((c // 8) * 8, MAXSEG * CPAD - 88)],
                            [segs[c % 8]], vals)

                @pl.when(k + 2 < nchunk)
                def _():
                    issue(k + 2, s)

            return 0

        lax.fori_loop(0, nchunk // 2, pair_body, 0)

        for sb in range(MAXSEG // L):
            v = hist[pl.ds(sb * L, L)]
            for ln in range(1, L):
                v = v + hist[pl.ds(ln * MAXSEG + sb * L, L)]
            cbuf[pl.ds(sb * L, L)] = v

        pltpu.sync_copy(acc, psums.at[wid])
        pltpu.sync_copy(cbuf, pcounts.at[wid])

    return body


def _tc_rows_body(seg_ref, feat_ref, sums_ref, cnt_ref):
    t = pl.program_id(1)
    n = ROWS_T * seg_ref.shape[2]
    seg = seg_ref[...].reshape(1, n)
    sids = lax.broadcasted_iota(jnp.int32, (MAXSEG, n), 0)
    oh = (sids == seg).astype(jnp.bfloat16)
    fb = feat_ref[...].reshape(feat_ref.shape[1], n).astype(jnp.bfloat16)
    part = lax.dot_general(oh, fb, (((1,), (1,)), ((), ())),
                           preferred_element_type=jnp.float32)
    ones_row = jnp.ones((1, n), jnp.bfloat16)
    cnt = lax.dot_general(ones_row, oh, (((1,), (1,)), ((), ())),
                          preferred_element_type=jnp.float32)

    @pl.when(t == 0)
    def _():
        sums_ref[0] = part
        cnt_ref[0] = cnt

    @pl.when(t != 0)
    def _():
        sums_ref[0] += part
        cnt_ref[0] += cnt


def _combine_body(ps_ref, pc_ref, ts_ref, tn_ref, cnx_ref, cny_ref, wx_ref,
                  wy_ref, bp_ref, inv_wh_ref, out_ref):
    sums = jnp.sum(ps_ref[...], axis=1)[..., :96] + ts_ref[...]
    counts = jnp.sum(pc_ref[...], axis=1) + tn_ref[..., 0, :]
    emb = sums / jnp.clip(counts, 1.0, None)[..., None]
    inv_w = inv_wh_ref[0, 0]
    inv_h = inv_wh_ref[0, 1]
    pos = (cnx_ref[...][..., None] * inv_w * wx_ref[...]
           + cny_ref[...][..., None] * inv_h * wy_ref[...]
           + bp_ref[...])
    out_ref[...] = emb + pos


def kernel(img, features, segments, centroid_coords, W_pos, b_pos):
    B, C, H, W = features.shape
    _, _, height, width = img.shape
    seg = segments.astype(jnp.int32)

    psums, pcounts = _build_sc_scatter(B, C, W)(features, seg)
    ps4 = psums.reshape(B, 32 // B, MAXSEG, CPAD)
    pc4 = pcounts.reshape(B, 32 // B, MAXSEG)

    nt = (H - H_SC) // ROWS_T
    row0 = H_SC // ROWS_T
    tc_sums, tc_cnt = pl.pallas_call(
        _tc_rows_body,
        grid=(B, nt),
        in_specs=[
            pl.BlockSpec((1, ROWS_T, W), lambda b, t: (b, row0 + t, 0)),
            pl.BlockSpec((1, C, ROWS_T, W), lambda b, t: (b, 0, row0 + t, 0)),
        ],
        out_specs=[
            pl.BlockSpec((1, MAXSEG, C), lambda b, t: (b, 0, 0)),
            pl.BlockSpec((1, 1, MAXSEG), lambda b, t: (b, 0, 0)),
        ],
        out_shape=[
            jax.ShapeDtypeStruct((B, MAXSEG, C), jnp.float32),
            jax.ShapeDtypeStruct((B, 1, MAXSEG), jnp.float32),
        ],
        compiler_params=pltpu.CompilerParams(
            dimension_semantics=("parallel", "arbitrary")),
    )(seg, features)

    cnx = centroid_coords[..., 0]
    cny = centroid_coords[..., 1]
    wx = W_pos[:, 0].reshape(1, 1, C)
    wy = W_pos[:, 1].reshape(1, 1, C)
    bp = b_pos.reshape(1, 1, C)
    inv_wh = jnp.array([[1.0 / float(width), 1.0 / float(height)]],
                       dtype=jnp.float32)

    out = pl.pallas_call(
        _combine_body,
        out_shape=jax.ShapeDtypeStruct((B, MAXSEG, C), jnp.float32),
    )(ps4, pc4, tc_sums, tc_cnt, cnx, cny, wx, wy, bp, inv_wh)
    return out


# speedup vs baseline: 13.1666x; 1.0601x over previous
import functools

import jax
import jax.numpy as jnp
from jax import lax
from jax.experimental import pallas as pl
from jax.experimental.pallas import tpu as pltpu
from jax.experimental.pallas import tpu_sc as plsc

MAXSEG = 256
L = 16
CPAD = 97
H_SC = 192
ROWS_T = 8


def _build_sc_scatter(B, C, W):
    NW = 32
    tecs_per_b = NW // B
    rows_per_tec = H_SC // tecs_per_b
    P = W
    nchunk = rows_per_tec
    groups = P // L
    mesh = plsc.VectorSubcoreMesh(core_axis_name="c", subcore_axis_name="s",
                                  num_cores=2, num_subcores=16)

    @functools.partial(
        pl.kernel,
        out_type=[
            jax.ShapeDtypeStruct((NW, MAXSEG * CPAD), jnp.float32),
            jax.ShapeDtypeStruct((NW, MAXSEG), jnp.float32),
        ],
        mesh=mesh,
        compiler_params=pltpu.CompilerParams(needs_layout_passes=False),
        scratch_types=[
            pltpu.VMEM((2, C, 1, P), jnp.float32),
            pltpu.VMEM((2, 1, P), jnp.int32),
            pltpu.VMEM((MAXSEG * CPAD,), jnp.float32),
            pltpu.VMEM((L * MAXSEG,), jnp.float32),
            pltpu.VMEM((MAXSEG,), jnp.float32),
            pltpu.SemaphoreType.DMA((2,)),
            pltpu.SemaphoreType.DMA((2,)),
        ],
    )
    def body(feat, seg, psums, pcounts, fbuf, sbuf, acc, hist, cbuf,
             sem_f, sem_s):
        wid = lax.axis_index("s") * 2 + lax.axis_index("c")
        b = wid // tecs_per_b
        slot = wid % tecs_per_b
        base_row = slot * rows_per_tec

        zf = jnp.zeros((L,), jnp.float32)
        onesf = jnp.ones((L,), jnp.float32)
        lane = lax.iota(jnp.int32, L)
        hist_off = lane * MAXSEG

        @plsc.parallel_loop(0, MAXSEG * CPAD // L, unroll=8)
        def zacc(i):
            acc[pl.ds(i * L, L)] = zf

        @plsc.parallel_loop(0, MAXSEG, unroll=8)
        def zhist(i):
            hist[pl.ds(i * L, L)] = zf

        def issue(k, s):
            r = base_row + k
            pltpu.async_copy(seg.at[b, pl.ds(r, 1), :], sbuf.at[s],
                             sem_s.at[s])
            pltpu.async_copy(feat.at[b, :, pl.ds(r, 1), :], fbuf.at[s],
                             sem_f.at[s])

        def drain(s):
            pltpu.make_async_copy(seg.at[b, pl.ds(base_row, 1), :],
                                  sbuf.at[s], sem_s.at[s]).wait()
            pltpu.make_async_copy(feat.at[b, :, pl.ds(base_row, 1), :],
                                  fbuf.at[s], sem_f.at[s]).wait()

        issue(0, 0)
        issue(1, 1)

        def pair_body(kk, _):
            for s in range(2):
                k = kk * 2 + s
                drain(s)

                @plsc.parallel_loop(0, groups, unroll=2)
                def group_body(g):
                    off = g * L
                    segv = sbuf[s, 0, pl.ds(off, L)]
                    plsc.addupdate_scatter(hist, [segv + hist_off], onesf)
                    segc = segv * CPAD
                    segs = [segc + r for r in range(8)]
                    for c in range(C):
                        vals = fbuf[s, c, 0, pl.ds(off, L)]
                        plsc.addupdate_scatter(
                            acc.at[pl.ds((c // 8) * 8, MAXSEG * CPAD - 88)],
                            [segs[c % 8]], vals)

                @pl.when(k + 2 < nchunk)
                def _():
                    issue(k + 2, s)

            return 0

        lax.fori_loop(0, nchunk // 2, pair_body, 0)

        for sb in range(MAXSEG // L):
            v = hist[pl.ds(sb * L, L)]
            for ln in range(1, L):
                v = v + hist[pl.ds(ln * MAXSEG + sb * L, L)]
            cbuf[pl.ds(sb * L, L)] = v

        pltpu.sync_copy(acc, psums.at[wid])
        pltpu.sync_copy(cbuf, pcounts.at[wid])

    return body


def _tc_rows_body(seg_ref, feat_ref, sums_ref, cnt_ref):
    t = pl.program_id(1)
    n = ROWS_T * seg_ref.shape[2]
    seg = seg_ref[...].reshape(1, n)
    sids = lax.broadcasted_iota(jnp.int32, (MAXSEG, n), 0)
    oh = (sids == seg).astype(jnp.bfloat16)
    fb = feat_ref[...].reshape(feat_ref.shape[1], n).astype(jnp.bfloat16)
    part = lax.dot_general(oh, fb, (((1,), (1,)), ((), ())),
                           preferred_element_type=jnp.float32)
    ones_row = jnp.ones((1, n), jnp.bfloat16)
    cnt = lax.dot_general(ones_row, oh, (((1,), (1,)), ((), ())),
                          preferred_element_type=jnp.float32)

    @pl.when(t == 0)
    def _():
        sums_ref[0] = part
        cnt_ref[0] = cnt

    @pl.when(t != 0)
    def _():
        sums_ref[0] += part
        cnt_ref[0] += cnt


def _combine_body(ps_ref, pc_ref, ts_ref, tn_ref, cnx_ref, cny_ref, wx_ref,
                  wy_ref, bp_ref, inv_wh_ref, out_ref):
    sums = jnp.sum(ps_ref[...], axis=1)[..., :96] + ts_ref[...]
    counts = jnp.sum(pc_ref[...], axis=1) + tn_ref[..., 0, :]
    emb = sums / jnp.clip(counts, 1.0, None)[..., None]
    inv_w = inv_wh_ref[0, 0]
    inv_h = inv_wh_ref[0, 1]
    pos = (cnx_ref[...][..., None] * inv_w * wx_ref[...]
           + cny_ref[...][..., None] * inv_h * wy_ref[...]
           + bp_ref[...])
    out_ref[...] = emb + pos


def kernel(img, features, segments, centroid_coords, W_pos, b_pos):
    B, C, H, W = features.shape
    _, _, height, width = img.shape
    seg = segments.astype(jnp.int32)

    psums, pcounts = _build_sc_scatter(B, C, W)(features, seg)
    ps4 = psums.reshape(B, 32 // B, MAXSEG, CPAD)
    pc4 = pcounts.reshape(B, 32 // B, MAXSEG)

    nt = (H - H_SC) // ROWS_T
    row0 = H_SC // ROWS_T
    tc_sums, tc_cnt = pl.pallas_call(
        _tc_rows_body,
        grid=(B, nt),
        in_specs=[
            pl.BlockSpec((1, ROWS_T, W), lambda b, t: (b, row0 + t, 0)),
            pl.BlockSpec((1, C, ROWS_T, W), lambda b, t: (b, 0, row0 + t, 0)),
        ],
        out_specs=[
            pl.BlockSpec((1, MAXSEG, C), lambda b, t: (b, 0, 0)),
            pl.BlockSpec((1, 1, MAXSEG), lambda b, t: (b, 0, 0)),
        ],
        out_shape=[
            jax.ShapeDtypeStruct((B, MAXSEG, C), jnp.float32),
            jax.ShapeDtypeStruct((B, 1, MAXSEG), jnp.float32),
        ],
        compiler_params=pltpu.CompilerParams(
            dimension_semantics=("parallel", "arbitrary")),
    )(seg, features)

    cnx = centroid_coords[..., 0]
    cny = centroid_coords[..., 1]
    wx = W_pos[:, 0].reshape(1, 1, C)
    wy = W_pos[:, 1].reshape(1, 1, C)
    bp = b_pos.reshape(1, 1, C)
    inv_wh = jnp.array([[1.0 / float(width), 1.0 / float(height)]],
                       dtype=jnp.float32)

    out = pl.pallas_call(
        _combine_body,
        out_shape=jax.ShapeDtypeStruct((B, MAXSEG, C), jnp.float32),
    )(ps4, pc4, tc_sums, tc_cnt, cnx, cny, wx, wy, bp, inv_wh)
    return out


# speedup vs baseline: 14.4023x; 1.0938x over previous
import functools

import jax
import jax.numpy as jnp
from jax import lax
from jax.experimental import pallas as pl
from jax.experimental.pallas import tpu as pltpu
from jax.experimental.pallas import tpu_sc as plsc

MAXSEG = 256
L = 16
CPAD = 97
H_SC = 160
ROWS_T = 8


def _build_sc_scatter(B, C, W):
    NW = 32
    tecs_per_b = NW // B
    rows_per_tec = H_SC // tecs_per_b
    P = W
    nchunk = rows_per_tec
    groups = P // L
    mesh = plsc.VectorSubcoreMesh(core_axis_name="c", subcore_axis_name="s",
                                  num_cores=2, num_subcores=16)

    @functools.partial(
        pl.kernel,
        out_type=[
            jax.ShapeDtypeStruct((NW, MAXSEG * CPAD), jnp.float32),
            jax.ShapeDtypeStruct((NW, MAXSEG), jnp.float32),
        ],
        mesh=mesh,
        compiler_params=pltpu.CompilerParams(needs_layout_passes=False),
        scratch_types=[
            pltpu.VMEM((2, C, 1, P), jnp.float32),
            pltpu.VMEM((2, 1, P), jnp.int32),
            pltpu.VMEM((MAXSEG * CPAD,), jnp.float32),
            pltpu.VMEM((L * MAXSEG,), jnp.float32),
            pltpu.VMEM((MAXSEG,), jnp.float32),
            pltpu.SemaphoreType.DMA((2,)),
            pltpu.SemaphoreType.DMA((2,)),
        ],
    )
    def body(feat, seg, psums, pcounts, fbuf, sbuf, acc, hist, cbuf,
             sem_f, sem_s):
        wid = lax.axis_index("s") * 2 + lax.axis_index("c")
        b = wid // tecs_per_b
        slot = wid % tecs_per_b
        base_row = slot * rows_per_tec

        zf = jnp.zeros((L,), jnp.float32)
        onesf = jnp.ones((L,), jnp.float32)
        lane = lax.iota(jnp.int32, L)
        hist_off = lane * MAXSEG

        @plsc.parallel_loop(0, MAXSEG * CPAD // L, unroll=8)
        def zacc(i):
            acc[pl.ds(i * L, L)] = zf

        @plsc.parallel_loop(0, MAXSEG, unroll=8)
        def zhist(i):
            hist[pl.ds(i * L, L)] = zf

        def issue(k, s):
            r = base_row + k
            pltpu.async_copy(seg.at[b, pl.ds(r, 1), :], sbuf.at[s],
                             sem_s.at[s])
            pltpu.async_copy(feat.at[b, :, pl.ds(r, 1), :], fbuf.at[s],
                             sem_f.at[s])

        def drain(s):
            pltpu.make_async_copy(seg.at[b, pl.ds(base_row, 1), :],
                                  sbuf.at[s], sem_s.at[s]).wait()
            pltpu.make_async_copy(feat.at[b, :, pl.ds(base_row, 1), :],
                                  fbuf.at[s], sem_f.at[s]).wait()

        issue(0, 0)
        issue(1, 1)

        def pair_body(kk, _):
            for s in range(2):
                k = kk * 2 + s
                drain(s)

                @plsc.parallel_loop(0, groups, unroll=2)
                def group_body(g):
                    off = g * L
                    segv = sbuf[s, 0, pl.ds(off, L)]
                    plsc.addupdate_scatter(hist, [segv + hist_off], onesf)
                    segc = segv * CPAD
                    segs = [segc + r for r in range(8)]
                    for c in range(C):
                        vals = fbuf[s, c, 0, pl.ds(off, L)]
                        plsc.addupdate_scatter(
                            acc.at[pl.ds((c // 8) * 8, MAXSEG * CPAD - 88)],
                            [segs[c % 8]], vals)

                @pl.when(k + 2 < nchunk)
                def _():
                    issue(k + 2, s)

            return 0

        lax.fori_loop(0, nchunk // 2, pair_body, 0)

        for sb in range(MAXSEG // L):
            v = hist[pl.ds(sb * L, L)]
            for ln in range(1, L):
                v = v + hist[pl.ds(ln * MAXSEG + sb * L, L)]
            cbuf[pl.ds(sb * L, L)] = v

        pltpu.sync_copy(acc, psums.at[wid])
        pltpu.sync_copy(cbuf, pcounts.at[wid])

    return body


def _tc_rows_body(seg_ref, feat_ref, sums_ref, cnt_ref):
    t = pl.program_id(1)
    n = ROWS_T * seg_ref.shape[2]
    seg = seg_ref[...].reshape(1, n)
    sids = lax.broadcasted_iota(jnp.int32, (MAXSEG, n), 0)
    oh = (sids == seg).astype(jnp.bfloat16)
    fb = feat_ref[...].reshape(feat_ref.shape[1], n).astype(jnp.bfloat16)
    part = lax.dot_general(oh, fb, (((1,), (1,)), ((), ())),
                           preferred_element_type=jnp.float32)
    ones_row = jnp.ones((1, n), jnp.bfloat16)
    cnt = lax.dot_general(ones_row, oh, (((1,), (1,)), ((), ())),
                          preferred_element_type=jnp.float32)

    @pl.when(t == 0)
    def _():
        sums_ref[0] = part
        cnt_ref[0] = cnt

    @pl.when(t != 0)
    def _():
        sums_ref[0] += part
        cnt_ref[0] += cnt


def _combine_body(ps_ref, pc_ref, ts_ref, tn_ref, cnx_ref, cny_ref, wx_ref,
                  wy_ref, bp_ref, inv_wh_ref, out_ref):
    sums = jnp.sum(ps_ref[...], axis=1)[..., :96] + ts_ref[...]
    counts = jnp.sum(pc_ref[...], axis=1) + tn_ref[..., 0, :]
    emb = sums / jnp.clip(counts, 1.0, None)[..., None]
    inv_w = inv_wh_ref[0, 0]
    inv_h = inv_wh_ref[0, 1]
    pos = (cnx_ref[...][..., None] * inv_w * wx_ref[...]
           + cny_ref[...][..., None] * inv_h * wy_ref[...]
           + bp_ref[...])
    out_ref[...] = emb + pos


def kernel(img, features, segments, centroid_coords, W_pos, b_pos):
    B, C, H, W = features.shape
    _, _, height, width = img.shape
    seg = segments.astype(jnp.int32)

    psums, pcounts = _build_sc_scatter(B, C, W)(features, seg)
    ps4 = psums.reshape(B, 32 // B, MAXSEG, CPAD)
    pc4 = pcounts.reshape(B, 32 // B, MAXSEG)

    nt = (H - H_SC) // ROWS_T
    row0 = H_SC // ROWS_T
    tc_sums, tc_cnt = pl.pallas_call(
        _tc_rows_body,
        grid=(B, nt),
        in_specs=[
            pl.BlockSpec((1, ROWS_T, W), lambda b, t: (b, row0 + t, 0)),
            pl.BlockSpec((1, C, ROWS_T, W), lambda b, t: (b, 0, row0 + t, 0)),
        ],
        out_specs=[
            pl.BlockSpec((1, MAXSEG, C), lambda b, t: (b, 0, 0)),
            pl.BlockSpec((1, 1, MAXSEG), lambda b, t: (b, 0, 0)),
        ],
        out_shape=[
            jax.ShapeDtypeStruct((B, MAXSEG, C), jnp.float32),
            jax.ShapeDtypeStruct((B, 1, MAXSEG), jnp.float32),
        ],
        compiler_params=pltpu.CompilerParams(
            dimension_semantics=("parallel", "arbitrary")),
    )(seg, features)

    cnx = centroid_coords[..., 0]
    cny = centroid_coords[..., 1]
    wx = W_pos[:, 0].reshape(1, 1, C)
    wy = W_pos[:, 1].reshape(1, 1, C)
    bp = b_pos.reshape(1, 1, C)
    inv_wh = jnp.array([[1.0 / float(width), 1.0 / float(height)]],
                       dtype=jnp.float32)

    out = pl.pallas_call(
        _combine_body,
        out_shape=jax.ShapeDtypeStruct((B, MAXSEG, C), jnp.float32),
    )(ps4, pc4, tc_sums, tc_cnt, cnx, cny, wx, wy, bp, inv_wh)
    return out
